# Initial kernel scaffold; baseline (speedup 1.0000x reference)
import jax
import jax.numpy as jnp
from jax.experimental import pallas as pl


def kernel(x, edge_index, W, b):
    raise NotImplementedError("write your pallas kernel here")


# speedup vs baseline: 119.6881x; 119.6881x over previous
import functools

import jax
import jax.numpy as jnp
from jax import lax
from jax.experimental import pallas as pl
from jax.experimental.pallas import tpu as pltpu
from jax.experimental.pallas import tpu_sc as plsc

N = 10000
E = 320000
D = 128
NW = 32
EPW = E // NW
LANES = 16

_mesh = plsc.VectorSubcoreMesh(core_axis_name="c", subcore_axis_name="s")
_sc_params = pltpu.CompilerParams(needs_layout_passes=False)


@functools.partial(
    pl.kernel,
    mesh=_mesh,
    out_type=jax.ShapeDtypeStruct((NW, N), jnp.float32),
    compiler_params=_sc_params,
    scratch_types=[
        pltpu.VMEM((EPW,), jnp.int32),
        pltpu.VMEM((N,), jnp.float32),
    ],
)
def _deg_kernel(dst_hbm, out_hbm, dst_v, acc_v):
    wid = lax.axis_index("c") * 16 + lax.axis_index("s")
    pltpu.sync_copy(dst_hbm.at[wid], dst_v)

    zeros = jnp.zeros((LANES,), jnp.float32)

    def init(i, carry):
        acc_v[pl.ds(i * LANES, LANES)] = zeros
        return carry

    lax.fori_loop(0, N // LANES, init, 0)

    ones = jnp.ones((LANES,), jnp.float32)

    def body(i, carry):
        dv = dst_v[pl.ds(i * LANES, LANES)]
        plsc.addupdate_scatter(acc_v, [dv], ones)
        return carry

    lax.fori_loop(0, EPW // LANES, body, 0)
    pltpu.sync_copy(acc_v, out_hbm.at[wid])


@functools.partial(
    pl.kernel,
    mesh=_mesh,
    out_type=jax.ShapeDtypeStruct((NW, N), jnp.float32),
    compiler_params=_sc_params,
    scratch_types=[
        pltpu.VMEM((EPW,), jnp.int32),
        pltpu.VMEM((EPW,), jnp.int32),
        pltpu.VMEM((N,), jnp.float32),
        pltpu.VMEM((N,), jnp.float32),
    ],
)
def _agg_kernel(src_hbm, dst_hbm, s_hbm, out_hbm, src_v, dst_v, s_v, acc_v):
    wid = lax.axis_index("c") * 16 + lax.axis_index("s")
    pltpu.sync_copy(src_hbm.at[wid], src_v)
    pltpu.sync_copy(dst_hbm.at[wid], dst_v)
    pltpu.sync_copy(s_hbm, s_v)

    zeros = jnp.zeros((LANES,), jnp.float32)

    def init(i, carry):
        acc_v[pl.ds(i * LANES, LANES)] = zeros
        return carry

    lax.fori_loop(0, N // LANES, init, 0)

    def body(i, carry):
        sv = plsc.load_gather(s_v, [src_v[pl.ds(i * LANES, LANES)]])
        plsc.addupdate_scatter(acc_v, [dst_v[pl.ds(i * LANES, LANES)]], sv)
        return carry

    lax.fori_loop(0, EPW // LANES, body, 0)
    pltpu.sync_copy(acc_v, out_hbm.at[wid])


def _tc1_body(x_ref, wt_ref, degp_ref, s_ref, dinv_ref):
    xw = lax.dot_general(
        wt_ref[...], x_ref[...], (((1,), (1,)), ((), ())),
        preferred_element_type=jnp.float32)
    deg = jnp.sum(degp_ref[...], axis=0, keepdims=True) + 1.0
    dinv = lax.rsqrt(deg)
    dinv_ref[...] = dinv
    s_ref[...] = xw * dinv


def _tc2_body(accp_ref, s_ref, dinv_ref, b_ref, o_ref):
    tot = jnp.sum(accp_ref[...], axis=0, keepdims=True) + s_ref[...]
    o_ref[...] = jnp.maximum(dinv_ref[...] * tot + b_ref[...], 0.0)


def kernel(x, edge_index, W, b):
    ei = edge_index.astype(jnp.int32)
    src = ei[0].reshape(NW, EPW)
    dst = ei[1].reshape(NW, EPW)
    wt = W.reshape(1, D)
    b2 = b.reshape(1, 1)

    degp = _deg_kernel(dst)

    s_row, dinv_row = pl.pallas_call(
        _tc1_body,
        out_shape=[
            jax.ShapeDtypeStruct((1, N), jnp.float32),
            jax.ShapeDtypeStruct((1, N), jnp.float32),
        ],
    )(x, wt, degp)

    accp = _agg_kernel(src, dst, s_row.reshape(N))

    out_row = pl.pallas_call(
        _tc2_body,
        out_shape=jax.ShapeDtypeStruct((1, N), jnp.float32),
    )(accp, s_row, dinv_row, b2)

    return out_row.reshape(N, 1)


# speedup vs baseline: 129.0261x; 1.0780x over previous
import functools

import jax
import jax.numpy as jnp
from jax import lax
from jax.experimental import pallas as pl
from jax.experimental.pallas import tpu as pltpu
from jax.experimental.pallas import tpu_sc as plsc

N = 10000
E = 320000
D = 128
NW = 32
EPW = E // NW
LANES = 16

_mesh = plsc.VectorSubcoreMesh(core_axis_name="c", subcore_axis_name="s")
_sc_params = pltpu.CompilerParams(needs_layout_passes=False)


@functools.partial(
    pl.kernel,
    mesh=_mesh,
    out_type=jax.ShapeDtypeStruct((NW, N), jnp.float32),
    compiler_params=_sc_params,
    scratch_types=[
        pltpu.VMEM((EPW,), jnp.int32),
        pltpu.VMEM((N,), jnp.float32),
    ],
)
def _deg_kernel(dst_hbm, out_hbm, dst_v, acc_v):
    wid = lax.axis_index("c") * 16 + lax.axis_index("s")
    pltpu.sync_copy(dst_hbm.at[wid], dst_v)

    zeros = jnp.zeros((LANES,), jnp.float32)

    def init(i, carry):
        acc_v[pl.ds(i * LANES, LANES)] = zeros
        return carry

    lax.fori_loop(0, N // LANES, init, 0, unroll=8)

    ones = jnp.ones((LANES,), jnp.float32)

    def body(i, carry):
        dv = dst_v[pl.ds(i * LANES, LANES)]
        plsc.addupdate_scatter(acc_v, [dv], ones)
        return carry

    lax.fori_loop(0, EPW // LANES, body, 0, unroll=8)
    pltpu.sync_copy(acc_v, out_hbm.at[wid])


@functools.partial(
    pl.kernel,
    mesh=_mesh,
    out_type=jax.ShapeDtypeStruct((NW, N), jnp.float32),
    compiler_params=_sc_params,
    scratch_types=[
        pltpu.VMEM((EPW,), jnp.int32),
        pltpu.VMEM((EPW,), jnp.int32),
        pltpu.VMEM((N,), jnp.float32),
        pltpu.VMEM((N,), jnp.float32),
    ],
)
def _agg_kernel(src_hbm, dst_hbm, s_hbm, out_hbm, src_v, dst_v, s_v, acc_v):
    wid = lax.axis_index("c") * 16 + lax.axis_index("s")
    pltpu.sync_copy(src_hbm.at[wid], src_v)
    pltpu.sync_copy(dst_hbm.at[wid], dst_v)
    pltpu.sync_copy(s_hbm, s_v)

    zeros = jnp.zeros((LANES,), jnp.float32)

    def init(i, carry):
        acc_v[pl.ds(i * LANES, LANES)] = zeros
        return carry

    lax.fori_loop(0, N // LANES, init, 0, unroll=8)

    def body(i, carry):
        sv = plsc.load_gather(s_v, [src_v[pl.ds(i * LANES, LANES)]])
        plsc.addupdate_scatter(acc_v, [dst_v[pl.ds(i * LANES, LANES)]], sv)
        return carry

    lax.fori_loop(0, EPW // LANES, body, 0, unroll=8)
    pltpu.sync_copy(acc_v, out_hbm.at[wid])


def _tc1_body(x_ref, wt_ref, degp_ref, s_ref, dinv_ref):
    xw = lax.dot_general(
        wt_ref[...], x_ref[...], (((1,), (1,)), ((), ())),
        preferred_element_type=jnp.float32)
    deg = jnp.sum(degp_ref[...], axis=0, keepdims=True) + 1.0
    dinv = lax.rsqrt(deg)
    dinv_ref[...] = dinv
    s_ref[...] = xw * dinv


def _tc2_body(accp_ref, s_ref, dinv_ref, b_ref, o_ref):
    tot = jnp.sum(accp_ref[...], axis=0, keepdims=True) + s_ref[...]
    o_ref[...] = jnp.maximum(dinv_ref[...] * tot + b_ref[...], 0.0)


def kernel(x, edge_index, W, b):
    ei = edge_index.astype(jnp.int32)
    src = ei[0].reshape(NW, EPW)
    dst = ei[1].reshape(NW, EPW)
    wt = W.reshape(1, D)
    b2 = b.reshape(1, 1)

    degp = _deg_kernel(dst)

    s_row, dinv_row = pl.pallas_call(
        _tc1_body,
        out_shape=[
            jax.ShapeDtypeStruct((1, N), jnp.float32),
            jax.ShapeDtypeStruct((1, N), jnp.float32),
        ],
    )(x, wt, degp)

    accp = _agg_kernel(src, dst, s_row.reshape(N))

    out_row = pl.pallas_call(
        _tc2_body,
        out_shape=jax.ShapeDtypeStruct((1, N), jnp.float32),
    )(accp, s_row, dinv_row, b2)

    return out_row.reshape(N, 1)


# speedup vs baseline: 137.0885x; 1.0625x over previous
import functools

import jax
import jax.numpy as jnp
from jax import lax
from jax.experimental import pallas as pl
from jax.experimental.pallas import tpu as pltpu
from jax.experimental.pallas import tpu_sc as plsc

N = 10000
E = 320000
D = 128
NW = 32
EPW = E // NW
LANES = 16

_mesh = plsc.VectorSubcoreMesh(core_axis_name="c", subcore_axis_name="s")
_sc_params = pltpu.CompilerParams(needs_layout_passes=False)


@functools.partial(
    pl.kernel,
    mesh=_mesh,
    out_type=jax.ShapeDtypeStruct((NW, N), jnp.float32),
    compiler_params=_sc_params,
    scratch_types=[
        pltpu.VMEM((EPW,), jnp.int32),
        pltpu.VMEM((N,), jnp.float32),
    ],
)
def _deg_kernel(dst_hbm, out_hbm, dst_v, acc_v):
    wid = lax.axis_index("c") * 16 + lax.axis_index("s")
    pltpu.sync_copy(dst_hbm.at[pl.ds(wid * EPW, EPW)], dst_v)

    zeros = jnp.zeros((LANES,), jnp.float32)

    def init(i, carry):
        acc_v[pl.ds(i * LANES, LANES)] = zeros
        return carry

    lax.fori_loop(0, N // LANES, init, 0, unroll=8)

    ones = jnp.ones((LANES,), jnp.float32)

    def body(i, carry):
        dv = dst_v[pl.ds(i * LANES, LANES)]
        plsc.addupdate_scatter(acc_v, [dv], ones)
        return carry

    lax.fori_loop(0, EPW // LANES, body, 0, unroll=8)
    pltpu.sync_copy(acc_v, out_hbm.at[wid])


@functools.partial(
    pl.kernel,
    mesh=_mesh,
    out_type=jax.ShapeDtypeStruct((NW, N), jnp.float32),
    compiler_params=_sc_params,
    scratch_types=[
        pltpu.VMEM((EPW,), jnp.int32),
        pltpu.VMEM((EPW,), jnp.int32),
        pltpu.VMEM((N,), jnp.float32),
        pltpu.VMEM((N,), jnp.float32),
    ],
)
def _agg_kernel(src_hbm, dst_hbm, s_hbm, out_hbm, src_v, dst_v, s_v, acc_v):
    wid = lax.axis_index("c") * 16 + lax.axis_index("s")
    pltpu.sync_copy(src_hbm.at[pl.ds(wid * EPW, EPW)], src_v)
    pltpu.sync_copy(dst_hbm.at[pl.ds(wid * EPW, EPW)], dst_v)
    pltpu.sync_copy(s_hbm, s_v)

    zeros = jnp.zeros((LANES,), jnp.float32)

    def init(i, carry):
        acc_v[pl.ds(i * LANES, LANES)] = zeros
        return carry

    lax.fori_loop(0, N // LANES, init, 0, unroll=8)

    def body(i, carry):
        sv = plsc.load_gather(s_v, [src_v[pl.ds(i * LANES, LANES)]])
        plsc.addupdate_scatter(acc_v, [dst_v[pl.ds(i * LANES, LANES)]], sv)
        return carry

    lax.fori_loop(0, EPW // LANES, body, 0, unroll=8)
    pltpu.sync_copy(acc_v, out_hbm.at[wid])


def _tc0_body(x_ref, wt_ref, xw_ref):
    xw_ref[...] = lax.dot_general(
        wt_ref[...], x_ref[...], (((1,), (1,)), ((), ())),
        preferred_element_type=jnp.float32)


def _tc1_body(xw_ref, degp_ref, s_ref, dinv_ref, s1_ref):
    deg = jnp.sum(degp_ref[...], axis=0, keepdims=True) + 1.0
    dinv = lax.rsqrt(deg)
    dinv_ref[...] = dinv
    s = xw_ref[...] * dinv
    s_ref[...] = s
    s1_ref[...] = s.reshape(N)


def _tc2_body(accp_ref, s_ref, dinv_ref, b_ref, o_ref):
    tot = jnp.sum(accp_ref[...], axis=0, keepdims=True) + s_ref[...]
    o_ref[...] = jnp.maximum(dinv_ref[...] * tot + b_ref[...], 0.0)


def kernel(x, edge_index, W, b):
    ei = edge_index.astype(jnp.int32)
    src = ei[0]
    dst = ei[1]
    wt = W.reshape(1, D)
    b2 = b.reshape(1, 1)

    xw_row = pl.pallas_call(
        _tc0_body,
        out_shape=jax.ShapeDtypeStruct((1, N), jnp.float32),
    )(x, wt)

    degp = _deg_kernel(dst)

    s_row, dinv_row, s1d = pl.pallas_call(
        _tc1_body,
        out_shape=[
            jax.ShapeDtypeStruct((1, N), jnp.float32),
            jax.ShapeDtypeStruct((1, N), jnp.float32),
            jax.ShapeDtypeStruct((N,), jnp.float32),
        ],
    )(xw_row, degp)

    accp = _agg_kernel(src, dst, s1d)

    out_row = pl.pallas_call(
        _tc2_body,
        out_shape=jax.ShapeDtypeStruct((1, N), jnp.float32),
    )(accp, s_row, dinv_row, b2)

    return out_row.reshape(N, 1)


# speedup vs baseline: 167.5090x; 1.2219x over previous
import functools

import jax
import jax.numpy as jnp
from jax import lax
from jax.experimental import pallas as pl
from jax.experimental.pallas import tpu as pltpu
from jax.experimental.pallas import tpu_sc as plsc

N = 10000
E = 320000
D = 128
NW = 32
EPW = E // NW
LANES = 16

_mesh = plsc.VectorSubcoreMesh(core_axis_name="c", subcore_axis_name="s")
_sc_params = pltpu.CompilerParams(needs_layout_passes=False)


@functools.partial(
    pl.kernel,
    mesh=_mesh,
    out_type=jax.ShapeDtypeStruct((NW, N), jnp.float32),
    compiler_params=_sc_params,
    scratch_types=[
        pltpu.VMEM((EPW,), jnp.int32),
        pltpu.VMEM((N,), jnp.float32),
    ],
)
def _deg_kernel(ei_hbm, out_hbm, dst_v, acc_v):
    wid = lax.axis_index("c") * 16 + lax.axis_index("s")
    pltpu.sync_copy(ei_hbm.at[pl.ds(E + wid * EPW, EPW)], dst_v)

    zeros = jnp.zeros((LANES,), jnp.float32)

    def init(i, carry):
        acc_v[pl.ds(i * LANES, LANES)] = zeros
        return carry

    lax.fori_loop(0, N // LANES, init, 0, unroll=8)

    ones = jnp.ones((LANES,), jnp.float32)

    def body(i, carry):
        dv = dst_v[pl.ds(i * LANES, LANES)]
        plsc.addupdate_scatter(acc_v, [dv], ones)
        return carry

    lax.fori_loop(0, EPW // LANES, body, 0, unroll=8)
    pltpu.sync_copy(acc_v, out_hbm.at[wid])


@functools.partial(
    pl.kernel,
    mesh=_mesh,
    out_type=jax.ShapeDtypeStruct((NW, N), jnp.float32),
    compiler_params=_sc_params,
    scratch_types=[
        pltpu.VMEM((EPW,), jnp.int32),
        pltpu.VMEM((EPW,), jnp.int32),
        pltpu.VMEM((N,), jnp.float32),
        pltpu.VMEM((N,), jnp.float32),
    ],
)
def _agg_kernel(ei_hbm, s_hbm, out_hbm, src_v, dst_v, s_v, acc_v):
    wid = lax.axis_index("c") * 16 + lax.axis_index("s")
    pltpu.sync_copy(ei_hbm.at[pl.ds(wid * EPW, EPW)], src_v)
    pltpu.sync_copy(ei_hbm.at[pl.ds(E + wid * EPW, EPW)], dst_v)
    pltpu.sync_copy(s_hbm, s_v)

    zeros = jnp.zeros((LANES,), jnp.float32)

    def init(i, carry):
        acc_v[pl.ds(i * LANES, LANES)] = zeros
        return carry

    lax.fori_loop(0, N // LANES, init, 0, unroll=8)

    def body(i, carry):
        sv = plsc.load_gather(s_v, [src_v[pl.ds(i * LANES, LANES)]])
        plsc.addupdate_scatter(acc_v, [dst_v[pl.ds(i * LANES, LANES)]], sv)
        return carry

    lax.fori_loop(0, EPW // LANES, body, 0, unroll=8)
    pltpu.sync_copy(acc_v, out_hbm.at[wid])


def _tc0_body(x_ref, wt_ref, xw_ref):
    xw_ref[...] = lax.dot_general(
        wt_ref[...], x_ref[...], (((1,), (1,)), ((), ())),
        preferred_element_type=jnp.float32)


def _tc1_body(xw_ref, degp_ref, s_ref, dinv_ref, s1_ref):
    deg = jnp.sum(degp_ref[...], axis=0, keepdims=True) + 1.0
    dinv = lax.rsqrt(deg)
    dinv_ref[...] = dinv
    s = xw_ref[...] * dinv
    s_ref[...] = s
    s1_ref[...] = s.reshape(N)


def _tc2_body(accp_ref, s_ref, dinv_ref, b_ref, o_ref):
    tot = jnp.sum(accp_ref[...], axis=0, keepdims=True) + s_ref[...]
    o_ref[...] = jnp.maximum(dinv_ref[...] * tot + b_ref[...], 0.0)


def kernel(x, edge_index, W, b):
    ei = edge_index.astype(jnp.int32).reshape(2 * E)
    wt = W.reshape(1, D)
    b2 = b.reshape(1, 1)

    xw_row = pl.pallas_call(
        _tc0_body,
        out_shape=jax.ShapeDtypeStruct((1, N), jnp.float32),
    )(x, wt)

    degp = _deg_kernel(ei)

    s_row, dinv_row, s1d = pl.pallas_call(
        _tc1_body,
        out_shape=[
            jax.ShapeDtypeStruct((1, N), jnp.float32),
            jax.ShapeDtypeStruct((1, N), jnp.float32),
            jax.ShapeDtypeStruct((N,), jnp.float32),
        ],
    )(xw_row, degp)

    accp = _agg_kernel(ei, s1d)

    out_row = pl.pallas_call(
        _tc2_body,
        out_shape=jax.ShapeDtypeStruct((1, N), jnp.float32),
    )(accp, s_row, dinv_row, b2)

    return out_row.reshape(N, 1)
